# baseline (speedup 1.0000x reference)
import functools

import jax
import jax.numpy as jnp
import numpy as np
from jax import lax
from jax.experimental import pallas as pl
from jax.experimental.pallas import tpu as pltpu
from jax.experimental.pallas import tpu_sc as plsc

N_NODES = 10000
N_EDGES = 320000
D = 128
HID = 256
G = 512
LAYERS = 5

NC = 2
NS = 16
NW = NC * NS
EPW = N_EDGES // NW
CH = 40
NCHUNK = EPW // CH
SUP = 50
NSUP = NCHUNK // SUP
PAIRS = SUP // 2

_ORDER = np.empty((N_EDGES,), np.int32)
_ORDER[0::2] = np.arange(N_EDGES // 2)
_ORDER[1::2] = np.arange(N_EDGES // 2) + N_EDGES // 2
RPT = 624
ZR = 208


@functools.lru_cache(maxsize=1)
def _sc_msgpass():
    mesh = plsc.VectorSubcoreMesh(core_axis_name="c", subcore_axis_name="s",
                                  num_cores=NC, num_subcores=NS)

    @functools.partial(
        pl.kernel,
        out_type=jax.ShapeDtypeStruct((NC, N_NODES, D), jnp.float32),
        mesh=mesh,
        scratch_types=[
            pltpu.VMEM((SUP, CH), jnp.int32),
            pltpu.VMEM((SUP, CH), jnp.int32),
            pltpu.VMEM((3, CH, D), jnp.float32),
            pltpu.VMEM((2, CH, D), jnp.uint32),
            pltpu.VMEM((2, CH, D), jnp.float32),
            pltpu.VMEM_SHARED((N_NODES, D), jnp.float32),
            pltpu.SemaphoreType.DMA((3,)),
            pltpu.SemaphoreType.DMA((2,)),
            pltpu.SemaphoreType.DMA((2,)),
        ],
    )
    def k(hin_hbm, e_hbm, srcs_hbm, dsts_hbm, out_hbm,
          src_v, dst_v, hbuf, epair, mbuf, agg_sh,
          gsem, ssem, epsem):
        cid = lax.axis_index("c")
        sid = lax.axis_index("s")
        wid = cid * NS + sid

        zero16 = jnp.zeros((16,), jnp.float32)

        def zrow(r, _):
            for c in range(D // 16):
                mbuf[0, r, pl.ds(c * 16, 16)] = zero16
            return 0

        lax.fori_loop(0, CH, zrow, 0)
        for j in range(RPT // 40):
            pltpu.sync_copy(mbuf.at[0],
                            agg_sh.at[pl.ds(sid * RPT + j * 40, 40)])
        pltpu.sync_copy(mbuf.at[0, pl.ds(0, 24)],
                        agg_sh.at[pl.ds(sid * RPT + 600, 24)])

        @pl.when(sid == NS - 1)
        def _():
            pltpu.sync_copy(mbuf.at[0, pl.ds(0, 16)],
                            agg_sh.at[pl.ds(NS * RPT, 16)])

        plsc.subcore_barrier()

        himask = jnp.full((16,), 0xFFFF0000, jnp.uint32)

        def issue_g(c):
            pltpu.async_copy(hin_hbm.at[src_v.at[c]], hbuf.at[c % 3],
                             gsem.at[c % 3])

        def issue_e(sup, p):
            base = pl.multiple_of(
                wid * (EPW // 2) + (sup * SUP + 2 * p) * (CH // 2), 8)
            pltpu.async_copy(e_hbm.at[pl.ds(base, CH)], epair.at[p % 2],
                             epsem.at[p % 2])

        def wait_s(b):
            pltpu.make_async_copy(mbuf.at[b], agg_sh.at[dst_v.at[0]],
                                  ssem.at[b]).wait()

        @pl.loop(0, NSUP)
        def _(sup):
            @pl.when(sup > 0)
            def _():
                wait_s(0)
                wait_s(1)
            pltpu.sync_copy(srcs_hbm.at[wid, sup], src_v)
            pltpu.sync_copy(dsts_hbm.at[wid, sup], dst_v)
            issue_e(sup, 0)
            issue_e(sup, 1)
            issue_g(0)
            issue_g(1)
            issue_g(2)

            @pl.loop(0, SUP)
            def _(c):
                b = c % 2
                hb = c % 3
                p = c // 2
                pb = p % 2
                erow0 = b * (CH // 2)

                @pl.when(c % 2 == 0)
                def _():
                    base = pl.multiple_of(
                        wid * (EPW // 2) + (sup * SUP + 2 * p) * (CH // 2), 8)
                    pltpu.make_async_copy(e_hbm.at[pl.ds(base, CH)],
                                          epair.at[pb], epsem.at[pb]).wait()

                pltpu.make_async_copy(hin_hbm.at[src_v.at[c]], hbuf.at[hb],
                                      gsem.at[hb]).wait()

                @pl.when(c >= 2)
                def _():
                    wait_s(b)

                @plsc.parallel_loop(0, CH // 2, 1, unroll=4)
                def _(rp):
                    for g4 in range(D // 16):
                        s = pl.ds(g4 * 16, 16)
                        ew = epair[pb, erow0 + rp, s]
                        e_lo = lax.bitcast_convert_type(ew << 16, jnp.float32)
                        e_hi = lax.bitcast_convert_type(ew & himask,
                                                        jnp.float32)
                        mbuf[b, 2 * rp, s] = jnp.maximum(
                            hbuf[hb, 2 * rp, s] + e_lo, 0.0)
                        mbuf[b, 2 * rp + 1, s] = jnp.maximum(
                            hbuf[hb, 2 * rp + 1, s] + e_hi, 0.0)

                pltpu.async_copy(mbuf.at[b], agg_sh.at[dst_v.at[c]],
                                 ssem.at[b], add=True)

                @pl.when(c + 3 < SUP)
                def _():
                    issue_g(c + 3)

                @pl.when((c % 2 == 1) & (p + 2 < PAIRS))
                def _():
                    issue_e(sup, p + 2)

        wait_s(0)
        wait_s(1)
        plsc.subcore_barrier()

        pltpu.sync_copy(agg_sh.at[pl.ds(sid * RPT, RPT)],
                        out_hbm.at[cid, pl.ds(sid * RPT, RPT)])

        @pl.when(sid == NS - 1)
        def _():
            pltpu.sync_copy(agg_sh.at[pl.ds(NS * RPT, 16)],
                            out_hbm.at[cid, pl.ds(NS * RPT, 16)])

    return k


RB = 2000
NRB = N_NODES // RB
EBH = 4000
NEBH = N_EDGES // 2 // EBH


def _p_body(batch_ref, p_ref):
    ids = lax.broadcasted_iota(jnp.int32, (RB, G), 1)
    p_ref[...] = (batch_ref[...] == ids).astype(jnp.float32)


def _build_p(batch):
    return pl.pallas_call(
        _p_body,
        grid=(NRB,),
        in_specs=[pl.BlockSpec((RB, 1), lambda i: (i, 0))],
        out_specs=pl.BlockSpec((RB, G), lambda i: (i, 0)),
        out_shape=jax.ShapeDtypeStruct((N_NODES, G), jnp.float32),
    )(batch.reshape(N_NODES, 1))


def _e_body(ea1_ref, ea2_ref, we_ref, ep_ref):
    lo = jnp.maximum(
        jnp.dot(ea1_ref[...], we_ref[...], preferred_element_type=jnp.float32),
        0.0).astype(jnp.bfloat16)
    hi = jnp.maximum(
        jnp.dot(ea2_ref[...], we_ref[...], preferred_element_type=jnp.float32),
        0.0).astype(jnp.bfloat16)
    lo_b = lax.bitcast_convert_type(lo, jnp.uint16).astype(jnp.uint32)
    hi_b = lax.bitcast_convert_type(hi, jnp.uint16).astype(jnp.uint32)
    ep_ref[...] = lo_b | (hi_b << 16)


def _bond_encode(edge_attr, we):
    return pl.pallas_call(
        _e_body,
        grid=(NEBH,),
        in_specs=[pl.BlockSpec((EBH, 16), lambda i: (i, 0)),
                  pl.BlockSpec((EBH, 16), lambda i: (i + NEBH, 0)),
                  pl.BlockSpec((16, D), lambda i: (0, 0))],
        out_specs=pl.BlockSpec((EBH, D), lambda i: (i, 0)),
        out_shape=jax.ShapeDtypeStruct((N_EDGES // 2, D), jnp.uint32),
    )(edge_attr, edge_attr, we)


def _u1_body(agg2_ref, hin_ref, p_ref, w1_ref, b1_ref, w2_ref, b2_ref,
             hnew_ref, pooled_ref):
    agg = agg2_ref[0] + agg2_ref[1]
    t = jnp.maximum(
        jnp.dot(agg, w1_ref[...], preferred_element_type=jnp.float32)
        + b1_ref[...], 0.0)
    out = jnp.dot(t, w2_ref[...], preferred_element_type=jnp.float32) \
        + b2_ref[...]
    hnew = hin_ref[...] + out
    hnew_ref[...] = hnew
    part = lax.dot_general(p_ref[...], hnew, (((0,), (0,)), ((), ())),
                           preferred_element_type=jnp.float32)

    @pl.when(pl.program_id(0) == 0)
    def _():
        pooled_ref[...] = jnp.zeros_like(pooled_ref)

    pooled_ref[...] += part


def _layer_update(agg2, hin, p, w1, b1, w2, b2):
    return pl.pallas_call(
        _u1_body,
        grid=(NRB,),
        in_specs=[
            pl.BlockSpec((NC, RB, D), lambda i: (0, i, 0)),
            pl.BlockSpec((RB, D), lambda i: (i, 0)),
            pl.BlockSpec((RB, G), lambda i: (i, 0)),
            pl.BlockSpec((D, HID), lambda i: (0, 0)),
            pl.BlockSpec((1, HID), lambda i: (0, 0)),
            pl.BlockSpec((HID, D), lambda i: (0, 0)),
            pl.BlockSpec((1, D), lambda i: (0, 0)),
        ],
        out_specs=[
            pl.BlockSpec((RB, D), lambda i: (i, 0)),
            pl.BlockSpec((G, D), lambda i: (0, 0)),
        ],
        out_shape=[
            jax.ShapeDtypeStruct((N_NODES, D), jnp.float32),
            jax.ShapeDtypeStruct((G, D), jnp.float32),
        ],
    )(agg2, hin, p, w1, b1.reshape(1, HID), w2, b2.reshape(1, D))


def _bcast_body(hnew_ref, p_ref, pooled_ref, virt_ref, wv_ref, bv_ref,
                hin_ref, virtnew_ref):
    virtnew = virt_ref[...] + jnp.maximum(
        jnp.dot(pooled_ref[...], wv_ref[...],
                preferred_element_type=jnp.float32) + bv_ref[...], 0.0)
    virtnew_ref[...] = virtnew
    hin_ref[...] = hnew_ref[...] + jnp.dot(
        p_ref[...], virtnew, preferred_element_type=jnp.float32)


def _virt_broadcast(hnew, p, pooled, virt, wv, bv):
    return pl.pallas_call(
        _bcast_body,
        grid=(NRB,),
        in_specs=[pl.BlockSpec((RB, D), lambda i: (i, 0)),
                  pl.BlockSpec((RB, G), lambda i: (i, 0)),
                  pl.BlockSpec((G, D), lambda i: (0, 0)),
                  pl.BlockSpec((G, D), lambda i: (0, 0)),
                  pl.BlockSpec((D, D), lambda i: (0, 0)),
                  pl.BlockSpec((1, D), lambda i: (0, 0))],
        out_specs=[pl.BlockSpec((RB, D), lambda i: (i, 0)),
                   pl.BlockSpec((G, D), lambda i: (0, 0))],
        out_shape=[jax.ShapeDtypeStruct((N_NODES, D), jnp.float32),
                   jax.ShapeDtypeStruct((G, D), jnp.float32)],
    )(hnew, p, pooled, virt, wv, bv.reshape(1, D))


def _norm_body(pooled_ref, hg_ref):
    p = pooled_ref[...]
    m = jnp.mean(p, axis=1, keepdims=True)
    v = jnp.mean((p - m) ** 2, axis=1, keepdims=True)
    hg_ref[...] = (p - m) * lax.rsqrt(v + 1e-5)


def _group_norm(pooled):
    return pl.pallas_call(
        _norm_body,
        out_shape=jax.ShapeDtypeStruct((G, D), jnp.float32),
    )(pooled)


def kernel(x, edge_index, edge_attr, batch, We, W1, b1, W2, b2, Wv, bv):
    src = edge_index[0][_ORDER].reshape(NW, NSUP, SUP, CH)
    dst = edge_index[1][_ORDER].reshape(NW, NSUP, SUP, CH)
    p = _build_p(batch)
    hin = x
    virt = jnp.zeros((G, D), jnp.float32)
    pooled = None
    for l in range(LAYERS):
        e = _bond_encode(edge_attr, We[l])
        agg2 = _sc_msgpass()(hin, e, src, dst)
        hnew, pooled = _layer_update(agg2, hin, p, W1[l], b1[l],
                                     W2[l], b2[l])
        if l < LAYERS - 1:
            hin, virt = _virt_broadcast(hnew, p, pooled, virt, Wv[l], bv[l])
    return _group_norm(pooled)


# speedup vs baseline: 1.0035x; 1.0035x over previous
import functools

import jax
import jax.numpy as jnp
import numpy as np
from jax import lax
from jax.experimental import pallas as pl
from jax.experimental.pallas import tpu as pltpu
from jax.experimental.pallas import tpu_sc as plsc

N_NODES = 10000
N_EDGES = 320000
D = 128
HID = 256
G = 512
LAYERS = 5

NC = 2
NS = 16
NW = NC * NS
EPW = N_EDGES // NW
CH = 40
NCHUNK = EPW // CH
SUP = 50
NSUP = NCHUNK // SUP
PAIRS = SUP // 2

_ORDER = np.empty((N_EDGES,), np.int32)
_ORDER[0::2] = np.arange(N_EDGES // 2)
_ORDER[1::2] = np.arange(N_EDGES // 2) + N_EDGES // 2
RPT = 624
ZR = 208


@functools.lru_cache(maxsize=1)
def _sc_msgpass():
    mesh = plsc.VectorSubcoreMesh(core_axis_name="c", subcore_axis_name="s",
                                  num_cores=NC, num_subcores=NS)

    @functools.partial(
        pl.kernel,
        out_type=jax.ShapeDtypeStruct((NC, N_NODES, D), jnp.float32),
        mesh=mesh,
        scratch_types=[
            pltpu.VMEM((SUP, CH), jnp.int32),
            pltpu.VMEM((SUP, CH), jnp.int32),
            pltpu.VMEM((3, CH, D), jnp.float32),
            pltpu.VMEM((2, CH, D), jnp.uint32),
            pltpu.VMEM((2, CH, D), jnp.float32),
            pltpu.VMEM_SHARED((N_NODES, D), jnp.float32),
            pltpu.SemaphoreType.DMA((3,)),
            pltpu.SemaphoreType.DMA((2,)),
            pltpu.SemaphoreType.DMA((2,)),
        ],
    )
    def k(hin_hbm, e_hbm, srcs_hbm, dsts_hbm, out_hbm,
          src_v, dst_v, hbuf, epair, mbuf, agg_sh,
          gsem, ssem, epsem):
        cid = lax.axis_index("c")
        sid = lax.axis_index("s")
        wid = cid * NS + sid

        zero16 = jnp.zeros((16,), jnp.float32)

        def zrow(r, _):
            for c in range(D // 16):
                mbuf[0, r, pl.ds(c * 16, 16)] = zero16
            return 0

        lax.fori_loop(0, CH, zrow, 0)
        for j in range(RPT // 40):
            pltpu.sync_copy(mbuf.at[0],
                            agg_sh.at[pl.ds(sid * RPT + j * 40, 40)])
        pltpu.sync_copy(mbuf.at[0, pl.ds(0, 24)],
                        agg_sh.at[pl.ds(sid * RPT + 600, 24)])

        @pl.when(sid == NS - 1)
        def _():
            pltpu.sync_copy(mbuf.at[0, pl.ds(0, 16)],
                            agg_sh.at[pl.ds(NS * RPT, 16)])

        plsc.subcore_barrier()

        himask = jnp.full((16,), 0xFFFF0000, jnp.uint32)

        def issue_g(c):
            pltpu.async_copy(hin_hbm.at[src_v.at[c]], hbuf.at[c % 3],
                             gsem.at[c % 3])

        def issue_e(sup, p):
            base = pl.multiple_of(
                wid * (EPW // 2) + (sup * SUP + 2 * p) * (CH // 2), 8)
            pltpu.async_copy(e_hbm.at[pl.ds(base, CH)], epair.at[p % 2],
                             epsem.at[p % 2])

        def wait_s(b):
            pltpu.make_async_copy(mbuf.at[b], agg_sh.at[dst_v.at[0]],
                                  ssem.at[b]).wait()

        @pl.loop(0, NSUP)
        def _(sup):
            @pl.when(sup > 0)
            def _():
                wait_s(0)
                wait_s(1)
            pltpu.sync_copy(srcs_hbm.at[wid, sup], src_v)
            pltpu.sync_copy(dsts_hbm.at[wid, sup], dst_v)
            issue_e(sup, 0)
            issue_e(sup, 1)
            issue_g(0)
            issue_g(1)
            issue_g(2)

            @pl.loop(0, SUP)
            def _(c):
                b = c % 2
                hb = c % 3
                p = c // 2
                pb = p % 2
                erow0 = b * (CH // 2)

                @pl.when(c % 2 == 0)
                def _():
                    base = pl.multiple_of(
                        wid * (EPW // 2) + (sup * SUP + 2 * p) * (CH // 2), 8)
                    pltpu.make_async_copy(e_hbm.at[pl.ds(base, CH)],
                                          epair.at[pb], epsem.at[pb]).wait()

                pltpu.make_async_copy(hin_hbm.at[src_v.at[c]], hbuf.at[hb],
                                      gsem.at[hb]).wait()

                @pl.when(c >= 2)
                def _():
                    wait_s(b)

                @plsc.parallel_loop(0, CH // 2, 1, unroll=2)
                def _(rp):
                    for g4 in range(D // 16):
                        s = pl.ds(g4 * 16, 16)
                        ew = epair[pb, erow0 + rp, s]
                        e_lo = lax.bitcast_convert_type(ew << 16, jnp.float32)
                        e_hi = lax.bitcast_convert_type(ew & himask,
                                                        jnp.float32)
                        mbuf[b, 2 * rp, s] = jnp.maximum(
                            hbuf[hb, 2 * rp, s] + e_lo, 0.0)
                        mbuf[b, 2 * rp + 1, s] = jnp.maximum(
                            hbuf[hb, 2 * rp + 1, s] + e_hi, 0.0)

                pltpu.async_copy(mbuf.at[b], agg_sh.at[dst_v.at[c]],
                                 ssem.at[b], add=True)

                @pl.when(c + 3 < SUP)
                def _():
                    issue_g(c + 3)

                @pl.when((c % 2 == 1) & (p + 2 < PAIRS))
                def _():
                    issue_e(sup, p + 2)

        wait_s(0)
        wait_s(1)
        plsc.subcore_barrier()

        pltpu.sync_copy(agg_sh.at[pl.ds(sid * RPT, RPT)],
                        out_hbm.at[cid, pl.ds(sid * RPT, RPT)])

        @pl.when(sid == NS - 1)
        def _():
            pltpu.sync_copy(agg_sh.at[pl.ds(NS * RPT, 16)],
                            out_hbm.at[cid, pl.ds(NS * RPT, 16)])

    return k


RB = 2000
NRB = N_NODES // RB
EBH = 4000
NEBH = N_EDGES // 2 // EBH


def _p_body(batch_ref, p_ref):
    ids = lax.broadcasted_iota(jnp.int32, (RB, G), 1)
    p_ref[...] = (batch_ref[...] == ids).astype(jnp.float32)


def _build_p(batch):
    return pl.pallas_call(
        _p_body,
        grid=(NRB,),
        in_specs=[pl.BlockSpec((RB, 1), lambda i: (i, 0))],
        out_specs=pl.BlockSpec((RB, G), lambda i: (i, 0)),
        out_shape=jax.ShapeDtypeStruct((N_NODES, G), jnp.float32),
    )(batch.reshape(N_NODES, 1))


def _e_body(ea1_ref, ea2_ref, we_ref, ep_ref):
    lo = jnp.maximum(
        jnp.dot(ea1_ref[...], we_ref[...], preferred_element_type=jnp.float32),
        0.0).astype(jnp.bfloat16)
    hi = jnp.maximum(
        jnp.dot(ea2_ref[...], we_ref[...], preferred_element_type=jnp.float32),
        0.0).astype(jnp.bfloat16)
    lo_b = lax.bitcast_convert_type(lo, jnp.uint16).astype(jnp.uint32)
    hi_b = lax.bitcast_convert_type(hi, jnp.uint16).astype(jnp.uint32)
    ep_ref[...] = lo_b | (hi_b << 16)


def _bond_encode(edge_attr, we):
    return pl.pallas_call(
        _e_body,
        grid=(NEBH,),
        in_specs=[pl.BlockSpec((EBH, 16), lambda i: (i, 0)),
                  pl.BlockSpec((EBH, 16), lambda i: (i + NEBH, 0)),
                  pl.BlockSpec((16, D), lambda i: (0, 0))],
        out_specs=pl.BlockSpec((EBH, D), lambda i: (i, 0)),
        out_shape=jax.ShapeDtypeStruct((N_EDGES // 2, D), jnp.uint32),
    )(edge_attr, edge_attr, we)


def _u1_body(agg2_ref, hin_ref, p_ref, w1_ref, b1_ref, w2_ref, b2_ref,
             hnew_ref, pooled_ref):
    agg = agg2_ref[0] + agg2_ref[1]
    t = jnp.maximum(
        jnp.dot(agg, w1_ref[...], preferred_element_type=jnp.float32)
        + b1_ref[...], 0.0)
    out = jnp.dot(t, w2_ref[...], preferred_element_type=jnp.float32) \
        + b2_ref[...]
    hnew = hin_ref[...] + out
    hnew_ref[...] = hnew
    part = lax.dot_general(p_ref[...], hnew, (((0,), (0,)), ((), ())),
                           preferred_element_type=jnp.float32)

    @pl.when(pl.program_id(0) == 0)
    def _():
        pooled_ref[...] = jnp.zeros_like(pooled_ref)

    pooled_ref[...] += part


def _layer_update(agg2, hin, p, w1, b1, w2, b2):
    return pl.pallas_call(
        _u1_body,
        grid=(NRB,),
        in_specs=[
            pl.BlockSpec((NC, RB, D), lambda i: (0, i, 0)),
            pl.BlockSpec((RB, D), lambda i: (i, 0)),
            pl.BlockSpec((RB, G), lambda i: (i, 0)),
            pl.BlockSpec((D, HID), lambda i: (0, 0)),
            pl.BlockSpec((1, HID), lambda i: (0, 0)),
            pl.BlockSpec((HID, D), lambda i: (0, 0)),
            pl.BlockSpec((1, D), lambda i: (0, 0)),
        ],
        out_specs=[
            pl.BlockSpec((RB, D), lambda i: (i, 0)),
            pl.BlockSpec((G, D), lambda i: (0, 0)),
        ],
        out_shape=[
            jax.ShapeDtypeStruct((N_NODES, D), jnp.float32),
            jax.ShapeDtypeStruct((G, D), jnp.float32),
        ],
    )(agg2, hin, p, w1, b1.reshape(1, HID), w2, b2.reshape(1, D))


def _bcast_body(hnew_ref, p_ref, pooled_ref, virt_ref, wv_ref, bv_ref,
                hin_ref, virtnew_ref):
    virtnew = virt_ref[...] + jnp.maximum(
        jnp.dot(pooled_ref[...], wv_ref[...],
                preferred_element_type=jnp.float32) + bv_ref[...], 0.0)
    virtnew_ref[...] = virtnew
    hin_ref[...] = hnew_ref[...] + jnp.dot(
        p_ref[...], virtnew, preferred_element_type=jnp.float32)


def _virt_broadcast(hnew, p, pooled, virt, wv, bv):
    return pl.pallas_call(
        _bcast_body,
        grid=(NRB,),
        in_specs=[pl.BlockSpec((RB, D), lambda i: (i, 0)),
                  pl.BlockSpec((RB, G), lambda i: (i, 0)),
                  pl.BlockSpec((G, D), lambda i: (0, 0)),
                  pl.BlockSpec((G, D), lambda i: (0, 0)),
                  pl.BlockSpec((D, D), lambda i: (0, 0)),
                  pl.BlockSpec((1, D), lambda i: (0, 0))],
        out_specs=[pl.BlockSpec((RB, D), lambda i: (i, 0)),
                   pl.BlockSpec((G, D), lambda i: (0, 0))],
        out_shape=[jax.ShapeDtypeStruct((N_NODES, D), jnp.float32),
                   jax.ShapeDtypeStruct((G, D), jnp.float32)],
    )(hnew, p, pooled, virt, wv, bv.reshape(1, D))


def _norm_body(pooled_ref, hg_ref):
    p = pooled_ref[...]
    m = jnp.mean(p, axis=1, keepdims=True)
    v = jnp.mean((p - m) ** 2, axis=1, keepdims=True)
    hg_ref[...] = (p - m) * lax.rsqrt(v + 1e-5)


def _group_norm(pooled):
    return pl.pallas_call(
        _norm_body,
        out_shape=jax.ShapeDtypeStruct((G, D), jnp.float32),
    )(pooled)


def kernel(x, edge_index, edge_attr, batch, We, W1, b1, W2, b2, Wv, bv):
    src = edge_index[0][_ORDER].reshape(NW, NSUP, SUP, CH)
    dst = edge_index[1][_ORDER].reshape(NW, NSUP, SUP, CH)
    p = _build_p(batch)
    hin = x
    virt = jnp.zeros((G, D), jnp.float32)
    pooled = None
    for l in range(LAYERS):
        e = _bond_encode(edge_attr, We[l])
        agg2 = _sc_msgpass()(hin, e, src, dst)
        hnew, pooled = _layer_update(agg2, hin, p, W1[l], b1[l],
                                     W2[l], b2[l])
        if l < LAYERS - 1:
            hin, virt = _virt_broadcast(hnew, p, pooled, virt, Wv[l], bv[l])
    return _group_norm(pooled)
